# baseline (speedup 1.0000x reference)
import functools
import math

import jax
import jax.numpy as jnp
from jax import lax
from jax.experimental import pallas as pl
from jax.experimental.pallas import tpu as pltpu
from jax.experimental.pallas import tpu_sc as plsc

D_MODEL = 64
SCALE = math.sqrt(D_MODEL)
NUM_CORES = 2
NUM_SUBCORES = 16
NUM_WORKERS = NUM_CORES * NUM_SUBCORES
SUB = 128
CHUNK = 512
SUBS_PER_CHUNK = CHUNK // SUB
LANES = 16


def _emb_body(x_hbm, table_hbm, out_hbm, idx_v, rows_v, gsem):
    b_total = out_hbm.shape[0]
    b_per_w = b_total // NUM_WORKERS
    n_chunks = b_per_w // CHUNK

    wid = lax.axis_index("s") * NUM_CORES + lax.axis_index("c")
    base_row = wid * b_per_w
    base_idx_row = wid * (b_per_w // SUB)

    def chunk_body(i, carry):
        pltpu.sync_copy(
            x_hbm.at[pl.ds(base_idx_row + i * SUBS_PER_CHUNK, SUBS_PER_CHUNK)],
            idx_v,
        )
        copies = [
            pltpu.async_copy(
                table_hbm.at[idx_v.at[j]],
                rows_v.at[pl.ds(j * SUB, SUB)],
                gsem,
            )
            for j in range(SUBS_PER_CHUNK)
        ]
        for c in copies:
            c.wait()

        def scale_body(r, c2):
            for cc in range(D_MODEL // LANES):
                rows_v[r, pl.ds(cc * LANES, LANES)] = (
                    rows_v[r, pl.ds(cc * LANES, LANES)] * SCALE
                )
            return c2

        lax.fori_loop(0, CHUNK, scale_body, 0, unroll=4)

        pltpu.sync_copy(rows_v, out_hbm.at[pl.ds(base_row + i * CHUNK, CHUNK)])
        return carry

    lax.fori_loop(0, n_chunks, chunk_body, 0)


def kernel(x, table):
    b0, b1 = x.shape
    b_total = b0 * b1
    x2d = x.reshape(b_total // SUB, SUB).astype(jnp.int32)

    mesh = plsc.VectorSubcoreMesh(
        core_axis_name="c",
        subcore_axis_name="s",
        num_cores=NUM_CORES,
        num_subcores=NUM_SUBCORES,
    )
    emb = pl.kernel(
        _emb_body,
        out_type=jax.ShapeDtypeStruct((b_total, D_MODEL), jnp.float32),
        mesh=mesh,
        scratch_types=[
            pltpu.VMEM((SUBS_PER_CHUNK, SUB), jnp.int32),
            pltpu.VMEM((CHUNK, D_MODEL), jnp.float32),
            pltpu.SemaphoreType.DMA,
        ],
        compiler_params=pltpu.CompilerParams(use_tc_tiling_on_sc=False),
    )
    out = emb(x2d, table)
    return out.reshape(b0, b1, D_MODEL)


# speedup vs baseline: 1.0188x; 1.0188x over previous
import functools
import math

import jax
import jax.numpy as jnp
from jax import lax
from jax.experimental import pallas as pl
from jax.experimental.pallas import tpu as pltpu
from jax.experimental.pallas import tpu_sc as plsc

D_MODEL = 64
SCALE = math.sqrt(D_MODEL)
NUM_CORES = 2
NUM_SUBCORES = 16
NUM_WORKERS = NUM_CORES * NUM_SUBCORES
SUB = 128
CHUNK = 512
SUBS_PER_CHUNK = CHUNK // SUB
LANES = 16


def _emb_body(x_hbm, table_hbm, out_hbm, idx_v, rows_v, gsem):
    b_total = out_hbm.shape[0]
    b_per_w = b_total // NUM_WORKERS
    n_chunks = b_per_w // CHUNK
    idx_rows = b_per_w // SUB

    wid = lax.axis_index("s") * NUM_CORES + lax.axis_index("c")
    base_row = wid * b_per_w
    base_idx_row = wid * idx_rows

    pltpu.sync_copy(x_hbm.at[pl.ds(base_idx_row, idx_rows)], idx_v)

    def chunk_body(i, carry):
        for j in range(SUBS_PER_CHUNK):
            pltpu.async_copy(
                table_hbm.at[idx_v.at[i * SUBS_PER_CHUNK + j]],
                rows_v.at[pl.ds(j * SUB, SUB)],
                gsem,
            )
        pltpu.make_async_copy(
            table_hbm.at[idx_v.at[0]], rows_v, gsem
        ).wait()

        @plsc.parallel_loop(0, CHUNK, 1, unroll=8)
        def scale_body(r):
            for cc in range(D_MODEL // LANES):
                rows_v[r, pl.ds(cc * LANES, LANES)] = (
                    rows_v[r, pl.ds(cc * LANES, LANES)] * SCALE
                )

        pltpu.sync_copy(rows_v, out_hbm.at[pl.ds(base_row + i * CHUNK, CHUNK)])
        return carry

    lax.fori_loop(0, n_chunks, chunk_body, 0)


def kernel(x, table):
    b0, b1 = x.shape
    b_total = b0 * b1
    x2d = x.reshape(b_total // SUB, SUB).astype(jnp.int32)

    mesh = plsc.VectorSubcoreMesh(
        core_axis_name="c",
        subcore_axis_name="s",
        num_cores=NUM_CORES,
        num_subcores=NUM_SUBCORES,
    )
    emb = pl.kernel(
        _emb_body,
        out_type=jax.ShapeDtypeStruct((b_total, D_MODEL), jnp.float32),
        mesh=mesh,
        scratch_types=[
            pltpu.VMEM((b_total // NUM_WORKERS // SUB, SUB), jnp.int32),
            pltpu.VMEM((CHUNK, D_MODEL), jnp.float32),
            pltpu.SemaphoreType.DMA,
        ],
        compiler_params=pltpu.CompilerParams(use_tc_tiling_on_sc=False),
    )
    out = emb(x2d, table)
    return out.reshape(b0, b1, D_MODEL)


# speedup vs baseline: 1.0848x; 1.0648x over previous
import functools
import math

import jax
import jax.numpy as jnp
from jax import lax
from jax.experimental import pallas as pl
from jax.experimental.pallas import tpu as pltpu
from jax.experimental.pallas import tpu_sc as plsc

D_MODEL = 64
SCALE = math.sqrt(D_MODEL)
NUM_CORES = 2
NUM_SUBCORES = 16
NUM_WORKERS = NUM_CORES * NUM_SUBCORES
SUB = 128
CHUNK = 512
SUBS_PER_CHUNK = CHUNK // SUB
NBUF = 3
LANES = 16


def _emb_body(x_hbm, table_hbm, out_hbm, idx_v, rows_v,
              g0, g1, g2, o0, o1, o2):
    gsems = (g0, g1, g2)
    osems = (o0, o1, o2)

    b_total = out_hbm.shape[0]
    b_per_w = b_total // NUM_WORKERS
    n_chunks = b_per_w // CHUNK
    idx_rows = b_per_w // SUB

    wid = lax.axis_index("s") * NUM_CORES + lax.axis_index("c")
    base_row = wid * b_per_w
    base_idx_row = wid * idx_rows

    def fire_gathers(c, b):
        for j in range(SUBS_PER_CHUNK):
            pltpu.async_copy(
                table_hbm.at[idx_v.at[c * SUBS_PER_CHUNK + j]],
                rows_v.at[b].at[pl.ds(j * SUB, SUB)],
                gsems[b],
            )

    def drain_gathers(b):
        pltpu.make_async_copy(
            table_hbm.at[idx_v.at[0]], rows_v.at[b], gsems[b]
        ).wait()

    def scale(b):
        @plsc.parallel_loop(0, CHUNK, 1, unroll=8)
        def scale_body(r):
            for cc in range(D_MODEL // LANES):
                rows_v[b, r, pl.ds(cc * LANES, LANES)] = (
                    rows_v[b, r, pl.ds(cc * LANES, LANES)] * SCALE
                )

    def fire_out(c, b):
        pltpu.async_copy(
            rows_v.at[b],
            out_hbm.at[pl.ds(base_row + c * CHUNK, CHUNK)],
            osems[b],
        )

    def wait_out(b):
        pltpu.make_async_copy(
            rows_v.at[b], out_hbm.at[pl.ds(0, CHUNK)], osems[b]
        ).wait()

    pltpu.sync_copy(x_hbm.at[pl.ds(base_idx_row, idx_rows)], idx_v)

    fire_gathers(0, 0)
    fire_gathers(1, 1)
    drain_gathers(0)
    scale(0)
    fire_out(0, 0)
    fire_gathers(2, 2)
    drain_gathers(1)
    scale(1)
    fire_out(1, 1)
    wait_out(0)
    fire_gathers(3, 0)

    def slot(c, b):
        drain_gathers(b)
        scale(b)
        fire_out(c, b)
        wait_out((b + 2) % NBUF)
        fire_gathers(c + 2, (b + 2) % NBUF)

    def group(h, carry):
        c = 2 + h * NBUF
        slot(c, 2)
        slot(c + 1, 0)
        slot(c + 2, 1)
        return carry

    n_groups = (n_chunks - 2 - NBUF) // NBUF
    lax.fori_loop(0, n_groups, group, 0)

    c_tail = 2 + n_groups * NBUF
    drain_gathers(2)
    scale(2)
    fire_out(c_tail, 2)
    wait_out(1)
    fire_gathers(c_tail + 2, 1)
    drain_gathers(0)
    scale(0)
    fire_out(c_tail + 1, 0)
    drain_gathers(1)
    scale(1)
    fire_out(c_tail + 2, 1)
    for b in range(NBUF):
        wait_out(b)


def kernel(x, table):
    b0, b1 = x.shape
    b_total = b0 * b1
    x2d = x.reshape(b_total // SUB, SUB).astype(jnp.int32)

    mesh = plsc.VectorSubcoreMesh(
        core_axis_name="c",
        subcore_axis_name="s",
        num_cores=NUM_CORES,
        num_subcores=NUM_SUBCORES,
    )
    emb = pl.kernel(
        _emb_body,
        out_type=jax.ShapeDtypeStruct((b_total, D_MODEL), jnp.float32),
        mesh=mesh,
        scratch_types=[
            pltpu.VMEM((b_total // NUM_WORKERS // SUB, SUB), jnp.int32),
            pltpu.VMEM((NBUF, CHUNK, D_MODEL), jnp.float32),
            pltpu.SemaphoreType.DMA,
            pltpu.SemaphoreType.DMA,
            pltpu.SemaphoreType.DMA,
            pltpu.SemaphoreType.DMA,
            pltpu.SemaphoreType.DMA,
            pltpu.SemaphoreType.DMA,
        ],
        compiler_params=pltpu.CompilerParams(use_tc_tiling_on_sc=False),
    )
    out = emb(x2d, table)
    return out.reshape(b0, b1, D_MODEL)
